# baseline (speedup 1.0000x reference)
import jax
import jax.numpy as jnp
from jax import lax
from jax.experimental import pallas as pl
from jax.experimental.pallas import tpu as pltpu
from jax.experimental.pallas import tpu_sc as plsc

_N = 10000
_E = 320000
_ETOT = _E + _N
_DIN = 128
_D = 64
_L = 10
_NC = 2
_NS = 16
_CHUNK = 128
_CHUNKS = 162
_EPW = _CHUNKS * _CHUNK
_EPAD = _NS * _EPW
_NPAD = 10240
_TROWS = _NPAD // _NS
_SCAP = 1536
_SCAPV = _SCAP // 16
_GCAP = 1792
_GCAPC = _GCAP // _CHUNK
_GTOT = _NS * _GCAP
_MROWS = _NS * _NS * _GCAP


def _sc_scan_body(src_hbm, dst_hbm, seg_src_hbm, seg_dst_hbm, seg_cnt_hbm,
                  src_v, dst_v, ssrc_v, sdst_v, cbuf_v):
    c = lax.axis_index("c")
    s = lax.axis_index("s")
    wid = c * _NS + s
    pltpu.sync_copy(src_hbm.at[pl.ds(wid * _EPW, _EPW)], src_v)
    pltpu.sync_copy(dst_hbm.at[pl.ds(wid * _EPW, _EPW)], dst_v)

    def scan_body(i, cnts):
        srcv = src_v[pl.ds(i * 16, 16)]
        dstv = dst_v[pl.ds(i * 16, 16)]
        new = []
        for t in range(_NS):
            lo = t * _TROWS
            m = (srcv >= lo) & (srcv < lo + _TROWS)
            csum = plsc.cumsum(m.astype(jnp.int32))
            pos = t * _SCAP + cnts[t] + csum - 1
            plsc.store_scatter(ssrc_v, [pos], srcv - lo, mask=m)
            plsc.store_scatter(sdst_v, [pos], dstv, mask=m)
            pc = plsc.all_reduce_population_count(m)
            new.append(jnp.minimum(cnts[t] + pc, _SCAP - 16))
        return tuple(new)

    zero16 = jnp.zeros((16,), jnp.int32)
    cnts = lax.fori_loop(0, _EPW // 16, scan_body, (zero16,) * _NS)
    for t in range(_NS):
        cbuf_v[pl.ds(t * 16, 16)] = cnts[t]
    pltpu.sync_copy(ssrc_v, seg_src_hbm.at[pl.ds(wid * _NS * _SCAP, _NS * _SCAP)])
    pltpu.sync_copy(sdst_v, seg_dst_hbm.at[pl.ds(wid * _NS * _SCAP, _NS * _SCAP)])
    pltpu.sync_copy(cbuf_v, seg_cnt_hbm.at[pl.ds(wid * _NS * 16, _NS * 16)])


def _sc_consol_body(seg_src_hbm, seg_dst_hbm, seg_cnt_hbm, fill_hbm,
                    srcl_hbm, dstl_hbm,
                    segs_v, segd_v, cbuf_v, srcl_v, dstl_v):
    c = lax.axis_index("c")
    t = lax.axis_index("s")
    wid = c * _NS + t
    pltpu.sync_copy(fill_hbm, srcl_v)
    pltpu.sync_copy(fill_hbm, dstl_v)
    lane = lax.iota(jnp.int32, 16)

    def seg_body(sl, gcnts):
        base = (c * _NS + sl) * _NS * _SCAP + t * _SCAP
        pltpu.sync_copy(seg_src_hbm.at[pl.ds(base, _SCAP)], segs_v)
        pltpu.sync_copy(seg_dst_hbm.at[pl.ds(base, _SCAP)], segd_v)
        pltpu.sync_copy(
            seg_cnt_hbm.at[pl.ds((c * _NS + sl) * _NS * 16 + t * 16, 16)], cbuf_v)
        cntspl = cbuf_v[pl.ds(0, 16)]

        def vreg_body(r, gc):
            posv = r * 16 + lane
            valid = posv < cntspl
            srcv = segs_v[pl.ds(r * 16, 16)]
            dstv = segd_v[pl.ds(r * 16, 16)]
            new = []
            for u in range(_NS):
                lo = u * _TROWS
                m = valid & (dstv >= lo) & (dstv < lo + _TROWS)
                csum = plsc.cumsum(m.astype(jnp.int32))
                pos = u * _GCAP + gc[u] + csum - 1
                plsc.store_scatter(srcl_v, [pos], srcv, mask=m)
                plsc.store_scatter(dstl_v, [pos], dstv - lo, mask=m)
                pc = plsc.all_reduce_population_count(m)
                new.append(jnp.minimum(gc[u] + pc, _GCAP - 16))
            return tuple(new)

        return lax.fori_loop(0, _SCAPV, vreg_body, gcnts)

    zero16 = jnp.zeros((16,), jnp.int32)
    lax.fori_loop(0, _NS, seg_body, (zero16,) * _NS)
    pltpu.sync_copy(srcl_v, srcl_hbm.at[pl.ds(wid * _GTOT, _GTOT)])
    pltpu.sync_copy(dstl_v, dstl_hbm.at[pl.ds(wid * _GTOT, _GTOT)])


def _sc_spmm_body(g_hbm, srcl_hbm, dstl_hbm, z_hbm, out_hbm, msg_hbm,
                  srcl_v, dstl_v, gsl_v, mbuf_v, msem):
    c = lax.axis_index("c")
    u_me = lax.axis_index("s")
    wid = c * _NS + u_me
    pltpu.sync_copy(srcl_hbm.at[pl.ds(wid * _GTOT, _GTOT)], srcl_v)
    g2 = g_hbm.at[c]
    m2 = msg_hbm.at[c]
    pltpu.sync_copy(g2.at[pl.ds(u_me * _TROWS, _TROWS)], gsl_v.at[pl.ds(0, _TROWS)])
    zv = jnp.zeros((16,), jnp.float32)
    for k in range(_D // 16):
        gsl_v[_TROWS, pl.ds(k * 16, 16)] = zv
    for t in range(_NS):
        pltpu.sync_copy(
            dstl_hbm.at[pl.ds((c * _NS + t) * _GTOT + u_me * _GCAP, _GCAP)],
            dstl_v.at[pl.ds(t * _GCAP, _GCAP)])
    lane = lax.iota(jnp.int32, 16)

    def asm_chunk(j, carry):
        for b in range(2):
            jj = 2 * j + b

            @pl.when(j > 0)
            def _():
                pltpu.make_async_copy(mbuf_v.at[b], m2.at[pl.ds(0, _CHUNK)],
                                      msem.at[b]).wait()

            base = jj * _CHUNK

            def sub_body(sub, inner):
                srcv = srcl_v[pl.ds(base + sub * 16, 16)]
                rowpos = sub * 16 + lane
                for f in range(_D):
                    fspl = jnp.full((16,), f, jnp.int32)
                    val = plsc.load_gather(gsl_v, [srcv, fspl])
                    plsc.store_scatter(mbuf_v.at[b], [rowpos, fspl], val)
                return inner

            lax.fori_loop(0, _CHUNK // 16, sub_body, 0)
            pltpu.async_copy(mbuf_v.at[b],
                             m2.at[pl.ds(wid * _GTOT + jj * _CHUNK, _CHUNK)],
                             msem.at[b])
        return carry

    lax.fori_loop(0, _GTOT // _CHUNK // 2, asm_chunk, 0)
    for b in range(2):
        pltpu.make_async_copy(mbuf_v.at[b], m2.at[pl.ds(0, _CHUNK)],
                              msem.at[b]).wait()
    plsc.subcore_barrier()

    pltpu.sync_copy(z_hbm, gsl_v)

    def acc_seg(t, carry):
        mbase = (c * _NS + t) * _GTOT + u_me * _GCAP

        def acc_chunk(j, inner):
            for b in range(2):
                jj = 2 * j + b
                pltpu.async_copy(m2.at[pl.ds(mbase + jj * _CHUNK, _CHUNK)],
                                 mbuf_v.at[b], msem.at[b])
            for b in range(2):
                jj = 2 * j + b
                pltpu.make_async_copy(m2.at[pl.ds(0, _CHUNK)], mbuf_v.at[b],
                                      msem.at[b]).wait()
                base = t * _GCAP + jj * _CHUNK
                mb = mbuf_v.at[b]

                def sub_body(q, inner2):
                    for eo in range(4):
                        e = q * 4 + eo
                        espl = jnp.full((16,), e, jnp.int32)
                        dspl = plsc.load_gather(
                            dstl_v, [jnp.full((16,), base + e, jnp.int32)])
                        for k in range(_D // 16):
                            klane = k * 16 + lane
                            val = plsc.load_gather(mb, [espl, klane])
                            plsc.addupdate_scatter(gsl_v, [dspl, klane], val)
                    return inner2

                lax.fori_loop(0, _CHUNK // 4, sub_body, 0)
            return inner

        lax.fori_loop(0, _GCAPC // 2, acc_chunk, 0)
        return carry

    lax.fori_loop(0, _NS, acc_seg, 0)
    pltpu.sync_copy(gsl_v.at[pl.ds(0, _TROWS)],
                    out_hbm.at[c].at[pl.ds(u_me * _TROWS, _TROWS)])


_sc_calls_cache = {}


def _sc_calls():
    if "scan" not in _sc_calls_cache:
        mesh = plsc.VectorSubcoreMesh(core_axis_name="c", subcore_axis_name="s",
                                      num_cores=_NC, num_subcores=_NS)
        params = pltpu.CompilerParams(use_tc_tiling_on_sc=False,
                                      needs_layout_passes=False)
        _sc_calls_cache["scan"] = pl.kernel(
            _sc_scan_body,
            out_type=(
                jax.ShapeDtypeStruct((_NC * _NS * _NS * _SCAP,), jnp.int32),
                jax.ShapeDtypeStruct((_NC * _NS * _NS * _SCAP,), jnp.int32),
                jax.ShapeDtypeStruct((_NC * _NS * _NS * 16,), jnp.int32),
            ),
            mesh=mesh,
            compiler_params=params,
            scratch_types=[
                pltpu.VMEM((_EPW,), jnp.int32),
                pltpu.VMEM((_EPW,), jnp.int32),
                pltpu.VMEM((_NS * _SCAP,), jnp.int32),
                pltpu.VMEM((_NS * _SCAP,), jnp.int32),
                pltpu.VMEM((_NS * 16,), jnp.int32),
            ],
        )
        _sc_calls_cache["consol"] = pl.kernel(
            _sc_consol_body,
            out_type=(
                jax.ShapeDtypeStruct((_NC * _NS * _GTOT,), jnp.int32),
                jax.ShapeDtypeStruct((_NC * _NS * _GTOT,), jnp.int32),
            ),
            mesh=mesh,
            compiler_params=params,
            scratch_types=[
                pltpu.VMEM((_SCAP,), jnp.int32),
                pltpu.VMEM((_SCAP,), jnp.int32),
                pltpu.VMEM((16,), jnp.int32),
                pltpu.VMEM((_GTOT,), jnp.int32),
                pltpu.VMEM((_GTOT,), jnp.int32),
            ],
        )
        _sc_calls_cache["spmm"] = pl.kernel(
            _sc_spmm_body,
            out_type=(
                jax.ShapeDtypeStruct((_NC, _NPAD, _D), jnp.float32),
                jax.ShapeDtypeStruct((_NC, _MROWS, _D), jnp.float32),
            ),
            mesh=mesh,
            compiler_params=params,
            scratch_types=[
                pltpu.VMEM((_GTOT,), jnp.int32),
                pltpu.VMEM((_GTOT,), jnp.int32),
                pltpu.VMEM((_TROWS + 8, _D), jnp.float32),
                pltpu.VMEM((2, _CHUNK, _D), jnp.float32),
                pltpu.SemaphoreType.DMA((2,)),
            ],
        )
    return _sc_calls_cache


def _rowmask():
    return lax.broadcasted_iota(jnp.int32, (_NPAD, 1), 0) < _N


def _dis(deg_ref, c):
    return lax.rsqrt(jnp.maximum(deg_ref[c, :, 0:1], 1.0))


def _tc_prep_body(xp_ref, w0_ref, deg_ref, g_ref):
    mask = _rowmask()
    for c in range(_NC):
        xw = jnp.dot(xp_ref[...], w0_ref[c], preferred_element_type=jnp.float32)
        g_ref[c] = jnp.where(mask, _dis(deg_ref, c) * xw, 0.0)


def _tc_step_body(s_ref, deg_ref, b_ref, w_ref, g_ref):
    mask = _rowmask()
    for c in range(_NC):
        dis = _dis(deg_ref, c)
        h = jnp.maximum(dis * s_ref[c] + b_ref[c], 0.0)
        g_ref[c] = jnp.where(
            mask, dis * jnp.dot(h, w_ref[c], preferred_element_type=jnp.float32), 0.0)


def _tc_final_body(s_ref, deg_ref, b_ref, watt_ref, out_ref):
    feats = []
    for c in range(_NC):
        feats.append(_dis(deg_ref, c) * s_ref[c] + b_ref[c])
    nf = jnp.concatenate(feats, axis=1)
    nrm = lax.rsqrt(jnp.sum(nf * nf, axis=1, keepdims=True))
    nfn = nf * nrm
    mask = _rowmask()
    nfn_m = jnp.where(mask, nfn, 0.0)
    mean = jnp.sum(nfn_m, axis=0, keepdims=True) * (1.0 / _N)
    ctx = jnp.tanh(jnp.dot(mean, watt_ref[...], preferred_element_type=jnp.float32))
    score = jax.nn.sigmoid(jnp.sum(nfn_m * ctx, axis=1, keepdims=True))
    gf = jnp.sum(jnp.where(mask, score * nfn_m, 0.0), axis=0, keepdims=True)
    out_ref[0] = jnp.concatenate(
        [nfn_m, jnp.broadcast_to(gf, (_NPAD, 2 * _D))], axis=1)


def _prep_call(xp, w0s, deg):
    return pl.pallas_call(
        _tc_prep_body,
        out_shape=jax.ShapeDtypeStruct((_NC, _NPAD, _D), jnp.float32),
    )(xp, w0s, deg)


def _step_call(sk, deg, bk, wk):
    return pl.pallas_call(
        _tc_step_body,
        out_shape=jax.ShapeDtypeStruct((_NC, _NPAD, _D), jnp.float32),
    )(sk, deg, bk, wk)


def _final_call(s9, deg, b9, watt):
    return pl.pallas_call(
        _tc_final_body,
        out_shape=jax.ShapeDtypeStruct((1, _NPAD, 4 * _D), jnp.float32),
    )(s9, deg, b9, watt)


def kernel(x, edge_index, batch, Wf0, bf0, Wf, bf, Wr0, br0, Wr, br, Watt):
    loopv = jnp.arange(_N, dtype=jnp.int32)
    padv = jnp.full((_EPAD - _ETOT,), jnp.int32(1 << 30), jnp.int32)
    a = jnp.concatenate([edge_index[0], loopv, padv])
    b = jnp.concatenate([edge_index[1], loopv, padv])
    src2 = jnp.stack([a, b]).reshape(-1)
    dst2 = jnp.stack([b, a]).reshape(-1)

    fill = jnp.full((_GTOT,), _TROWS, jnp.int32)
    zslab = jnp.zeros((_TROWS + 8, _D), jnp.float32)
    rmask = (jnp.arange(_NPAD) < _N).astype(jnp.float32)[:, None]
    ones_g = jnp.broadcast_to(rmask, (_NPAD, _D))[None] * jnp.ones((_NC, 1, 1), jnp.float32)
    xp = jnp.pad(x, ((0, _NPAD - _N), (0, 0)))

    w0s = jnp.stack([Wf0, Wr0])
    wks = jnp.stack([Wf, Wr])
    b0 = jnp.stack([bf0, br0])
    bks = jnp.stack([bf, br])

    sc = _sc_calls()
    seg_src, seg_dst, seg_cnt = sc["scan"](src2, dst2)
    srcl, dstl = sc["consol"](seg_src, seg_dst, seg_cnt, fill)

    deg, _ = sc["spmm"](ones_g, srcl, dstl, zslab)
    g = _prep_call(xp, w0s, deg)
    for k in range(_L - 1):
        sk, _ = sc["spmm"](g, srcl, dstl, zslab)
        bk = b0 if k == 0 else bks[:, k - 1]
        g = _step_call(sk, deg, bk, wks[:, k])
    s9, _ = sc["spmm"](g, srcl, dstl, zslab)
    out = _final_call(s9, deg, bks[:, _L - 2], Watt)
    return out[:, :_N, :]


# speedup vs baseline: 1.8474x; 1.8474x over previous
import jax
import jax.numpy as jnp
from jax import lax
from jax.experimental import pallas as pl
from jax.experimental.pallas import tpu as pltpu
from jax.experimental.pallas import tpu_sc as plsc

_N = 10000
_E = 320000
_ETOT = _E + _N
_DIN = 128
_D = 64
_L = 10
_NC = 2
_NS = 16
_CHUNK = 128
_CHUNKS = 162
_EPW = _CHUNKS * _CHUNK
_EPAD = _NS * _EPW
_NPAD = 10240
_TROWS = _NPAD // _NS
_SCAP = 1536
_SCAPV = _SCAP // 16
_NW = 4
_WROWS = _NPAD // _NW
_WRPT = _WROWS // _NS
_LCAPW = 6144
_LCAPWC = _LCAPW // _CHUNK
_LTOT = _NW * _LCAPW
_NBUF = 6


def _sc_scan_body(src_hbm, dst_hbm, seg_src_hbm, seg_dst_hbm, seg_cnt_hbm,
                  src_v, dst_v, ssrc_v, sdst_v, cbuf_v):
    c = lax.axis_index("c")
    s = lax.axis_index("s")
    wid = c * _NS + s
    pltpu.sync_copy(src_hbm.at[pl.ds(wid * _EPW, _EPW)], src_v)
    pltpu.sync_copy(dst_hbm.at[pl.ds(wid * _EPW, _EPW)], dst_v)

    def scan_body(i, cnts):
        srcv = src_v[pl.ds(i * 16, 16)]
        dstv = dst_v[pl.ds(i * 16, 16)]
        new = []
        for t in range(_NS):
            lo = t * _TROWS
            m = (srcv >= lo) & (srcv < lo + _TROWS)
            csum = plsc.cumsum(m.astype(jnp.int32))
            pos = t * _SCAP + cnts[t] + csum - 1
            plsc.store_scatter(ssrc_v, [pos], srcv, mask=m)
            plsc.store_scatter(sdst_v, [pos], dstv, mask=m)
            pc = plsc.all_reduce_population_count(m)
            new.append(jnp.minimum(cnts[t] + pc, _SCAP - 16))
        return tuple(new)

    zero16 = jnp.zeros((16,), jnp.int32)
    cnts = lax.fori_loop(0, _EPW // 16, scan_body, (zero16,) * _NS)
    for t in range(_NS):
        cbuf_v[pl.ds(t * 16, 16)] = cnts[t]
    pltpu.sync_copy(ssrc_v, seg_src_hbm.at[pl.ds(wid * _NS * _SCAP, _NS * _SCAP)])
    pltpu.sync_copy(sdst_v, seg_dst_hbm.at[pl.ds(wid * _NS * _SCAP, _NS * _SCAP)])
    pltpu.sync_copy(cbuf_v, seg_cnt_hbm.at[pl.ds(wid * _NS * 16, _NS * 16)])


def _sc_consol_body(seg_src_hbm, seg_dst_hbm, seg_cnt_hbm, fsrc_hbm, fdst_hbm,
                    srcl_hbm, dstl_hbm,
                    segs_v, segd_v, cbuf_v, srcl_v, dstl_v):
    c = lax.axis_index("c")
    t = lax.axis_index("s")
    wid = c * _NS + t
    pltpu.sync_copy(fsrc_hbm, srcl_v)
    pltpu.sync_copy(fdst_hbm, dstl_v)
    lane = lax.iota(jnp.int32, 16)

    def seg_body(sl, gcnts):
        base = (c * _NS + sl) * _NS * _SCAP + t * _SCAP
        pltpu.sync_copy(seg_src_hbm.at[pl.ds(base, _SCAP)], segs_v)
        pltpu.sync_copy(seg_dst_hbm.at[pl.ds(base, _SCAP)], segd_v)
        pltpu.sync_copy(
            seg_cnt_hbm.at[pl.ds((c * _NS + sl) * _NS * 16 + t * 16, 16)], cbuf_v)
        cntspl = cbuf_v[pl.ds(0, 16)]

        def vreg_body(r, gc):
            posv = r * 16 + lane
            valid = posv < cntspl
            srcv = segs_v[pl.ds(r * 16, 16)]
            dstv = segd_v[pl.ds(r * 16, 16)]
            new = []
            for w in range(_NW):
                lo = w * _WROWS
                m = valid & (dstv >= lo) & (dstv < lo + _WROWS)
                csum = plsc.cumsum(m.astype(jnp.int32))
                pos = w * _LCAPW + gc[w] + csum - 1
                plsc.store_scatter(srcl_v, [pos], srcv, mask=m)
                plsc.store_scatter(dstl_v, [pos], dstv - lo, mask=m)
                pc = plsc.all_reduce_population_count(m)
                new.append(jnp.minimum(gc[w] + pc, _LCAPW - 16))
            return tuple(new)

        return lax.fori_loop(0, _SCAPV, vreg_body, gcnts)

    zero16 = jnp.zeros((16,), jnp.int32)
    lax.fori_loop(0, _NS, seg_body, (zero16,) * _NW)
    pltpu.sync_copy(srcl_v, srcl_hbm.at[pl.ds(wid * _LTOT, _LTOT)])
    pltpu.sync_copy(dstl_v, dstl_hbm.at[pl.ds(wid * _LTOT, _LTOT)])


def _sc_spmm_body(g_hbm, srcl_hbm, dstl_hbm, z_hbm, out_hbm,
                  srcl_v, dstl_v, rows_v, zbuf_v, obuf_v, acc_sh, gsem, ssem):
    c = lax.axis_index("c")
    s = lax.axis_index("s")
    r0 = s * _WRPT
    pltpu.sync_copy(srcl_hbm.at[c, s], srcl_v)
    pltpu.sync_copy(dstl_hbm.at[c, s], dstl_v)
    pltpu.sync_copy(z_hbm, zbuf_v)
    g2 = g_hbm.at[c]
    gdum = g2.at[pl.ds(0, _CHUNK)]
    adum = acc_sh.at[pl.ds(0, _CHUNK)]
    pltpu.sync_copy(zbuf_v, acc_sh.at[pl.ds(r0, _WRPT)])
    plsc.subcore_barrier()
    nr = _LCAPWC // _NBUF
    for w in range(_NW):
        for b in range(_NBUF):
            pltpu.async_copy(g2.at[srcl_v.at[w, b]], rows_v.at[b], gsem.at[b])

        def round_body(r, carry):
            for b in range(_NBUF):
                pltpu.make_async_copy(gdum, rows_v.at[b], gsem.at[b]).wait()
                pltpu.async_copy(rows_v.at[b],
                                 acc_sh.at[dstl_v.at[w, r * _NBUF + b]],
                                 ssem.at[b], add=True)

            @pl.when(r + 1 < nr)
            def _():
                for b in range(_NBUF):
                    pltpu.make_async_copy(rows_v.at[b], adum, ssem.at[b]).wait()
                    pltpu.async_copy(g2.at[srcl_v.at[w, (r + 1) * _NBUF + b]],
                                     rows_v.at[b], gsem.at[b])

            return carry

        lax.fori_loop(0, nr, round_body, 0)
        for b in range(_NBUF):
            pltpu.make_async_copy(rows_v.at[b], adum, ssem.at[b]).wait()
        plsc.subcore_barrier()
        pltpu.sync_copy(acc_sh.at[pl.ds(r0, _WRPT)], obuf_v)
        pltpu.sync_copy(obuf_v, out_hbm.at[c].at[pl.ds(w * _WROWS + r0, _WRPT)])
        if w + 1 < _NW:
            pltpu.sync_copy(zbuf_v, acc_sh.at[pl.ds(r0, _WRPT)])
            plsc.subcore_barrier()


_sc_calls_cache = {}


def _sc_calls():
    if "scan" not in _sc_calls_cache:
        mesh = plsc.VectorSubcoreMesh(core_axis_name="c", subcore_axis_name="s",
                                      num_cores=_NC, num_subcores=_NS)
        params = pltpu.CompilerParams(use_tc_tiling_on_sc=False,
                                      needs_layout_passes=False)
        _sc_calls_cache["scan"] = pl.kernel(
            _sc_scan_body,
            out_type=(
                jax.ShapeDtypeStruct((_NC * _NS * _NS * _SCAP,), jnp.int32),
                jax.ShapeDtypeStruct((_NC * _NS * _NS * _SCAP,), jnp.int32),
                jax.ShapeDtypeStruct((_NC * _NS * _NS * 16,), jnp.int32),
            ),
            mesh=mesh,
            compiler_params=params,
            scratch_types=[
                pltpu.VMEM((_EPW,), jnp.int32),
                pltpu.VMEM((_EPW,), jnp.int32),
                pltpu.VMEM((_NS * _SCAP,), jnp.int32),
                pltpu.VMEM((_NS * _SCAP,), jnp.int32),
                pltpu.VMEM((_NS * 16,), jnp.int32),
            ],
        )
        _sc_calls_cache["consol"] = pl.kernel(
            _sc_consol_body,
            out_type=(
                jax.ShapeDtypeStruct((_NC * _NS * _LTOT,), jnp.int32),
                jax.ShapeDtypeStruct((_NC * _NS * _LTOT,), jnp.int32),
            ),
            mesh=mesh,
            compiler_params=params,
            scratch_types=[
                pltpu.VMEM((_SCAP,), jnp.int32),
                pltpu.VMEM((_SCAP,), jnp.int32),
                pltpu.VMEM((16,), jnp.int32),
                pltpu.VMEM((_LTOT,), jnp.int32),
                pltpu.VMEM((_LTOT,), jnp.int32),
            ],
        )
        _sc_calls_cache["spmm"] = pl.kernel(
            _sc_spmm_body,
            out_type=jax.ShapeDtypeStruct((_NC, _NPAD, _D), jnp.float32),
            mesh=mesh,
            compiler_params=params,
            scratch_types=[
                pltpu.VMEM((_NW, _LCAPWC, _CHUNK), jnp.int32),
                pltpu.VMEM((_NW, _LCAPWC, _CHUNK), jnp.int32),
                pltpu.VMEM((_NBUF, _CHUNK, _D), jnp.float32),
                pltpu.VMEM((_WRPT, _D), jnp.float32),
                pltpu.VMEM((_WRPT, _D), jnp.float32),
                pltpu.VMEM_SHARED((_WROWS, _D), jnp.float32),
                pltpu.SemaphoreType.DMA((_NBUF,)),
                pltpu.SemaphoreType.DMA((_NBUF,)),
            ],
        )
    return _sc_calls_cache


def _rowmask():
    return lax.broadcasted_iota(jnp.int32, (_NPAD, 1), 0) < _N


def _dis(deg_ref, c):
    return lax.rsqrt(jnp.maximum(deg_ref[c, :, 0:1], 1.0))


def _tc_prep_body(xp_ref, w0_ref, deg_ref, g_ref):
    mask = _rowmask()
    for c in range(_NC):
        xw = jnp.dot(xp_ref[...], w0_ref[c], preferred_element_type=jnp.float32)
        g_ref[c] = jnp.where(mask, _dis(deg_ref, c) * xw, 0.0)


def _tc_step_body(s_ref, deg_ref, b_ref, w_ref, g_ref):
    mask = _rowmask()
    for c in range(_NC):
        dis = _dis(deg_ref, c)
        h = jnp.maximum(dis * s_ref[c] + b_ref[c], 0.0)
        g_ref[c] = jnp.where(
            mask, dis * jnp.dot(h, w_ref[c], preferred_element_type=jnp.float32), 0.0)


def _tc_final_body(s_ref, deg_ref, b_ref, watt_ref, out_ref):
    feats = []
    for c in range(_NC):
        feats.append(_dis(deg_ref, c) * s_ref[c] + b_ref[c])
    nf = jnp.concatenate(feats, axis=1)
    nrm = lax.rsqrt(jnp.sum(nf * nf, axis=1, keepdims=True))
    nfn = nf * nrm
    mask = _rowmask()
    nfn_m = jnp.where(mask, nfn, 0.0)
    mean = jnp.sum(nfn_m, axis=0, keepdims=True) * (1.0 / _N)
    ctx = jnp.tanh(jnp.dot(mean, watt_ref[...], preferred_element_type=jnp.float32))
    score = jax.nn.sigmoid(jnp.sum(nfn_m * ctx, axis=1, keepdims=True))
    gf = jnp.sum(jnp.where(mask, score * nfn_m, 0.0), axis=0, keepdims=True)
    out_ref[0] = jnp.concatenate(
        [nfn_m, jnp.broadcast_to(gf, (_NPAD, 2 * _D))], axis=1)


def _prep_call(xp, w0s, deg):
    return pl.pallas_call(
        _tc_prep_body,
        out_shape=jax.ShapeDtypeStruct((_NC, _NPAD, _D), jnp.float32),
    )(xp, w0s, deg)


def _step_call(sk, deg, bk, wk):
    return pl.pallas_call(
        _tc_step_body,
        out_shape=jax.ShapeDtypeStruct((_NC, _NPAD, _D), jnp.float32),
    )(sk, deg, bk, wk)


def _final_call(s9, deg, b9, watt):
    return pl.pallas_call(
        _tc_final_body,
        out_shape=jax.ShapeDtypeStruct((1, _NPAD, 4 * _D), jnp.float32),
    )(s9, deg, b9, watt)


def kernel(x, edge_index, batch, Wf0, bf0, Wf, bf, Wr0, br0, Wr, br, Watt):
    loopv = jnp.arange(_N, dtype=jnp.int32)
    padv = jnp.full((_EPAD - _ETOT,), jnp.int32(1 << 30), jnp.int32)
    a = jnp.concatenate([edge_index[0], loopv, padv])
    b = jnp.concatenate([edge_index[1], loopv, padv])
    src2 = jnp.stack([a, b]).reshape(-1)
    dst2 = jnp.stack([b, a]).reshape(-1)

    fsrc = jnp.full((_LTOT,), _N, jnp.int32)
    fdst = jnp.zeros((_LTOT,), jnp.int32)
    zwin = jnp.zeros((_WRPT, _D), jnp.float32)
    rmask = (jnp.arange(_NPAD) < _N).astype(jnp.float32)[:, None]
    ones_g = jnp.broadcast_to(rmask, (_NPAD, _D))[None] * jnp.ones((_NC, 1, 1), jnp.float32)
    xp = jnp.pad(x, ((0, _NPAD - _N), (0, 0)))

    w0s = jnp.stack([Wf0, Wr0])
    wks = jnp.stack([Wf, Wr])
    b0 = jnp.stack([bf0, br0])
    bks = jnp.stack([bf, br])

    sc = _sc_calls()
    seg_src, seg_dst, seg_cnt = sc["scan"](src2, dst2)
    srcl, dstl = sc["consol"](seg_src, seg_dst, seg_cnt, fsrc, fdst)
    srcl = srcl.reshape(_NC, _NS, _NW, _LCAPWC, _CHUNK)
    dstl = dstl.reshape(_NC, _NS, _NW, _LCAPWC, _CHUNK)

    deg = sc["spmm"](ones_g, srcl, dstl, zwin)
    g = _prep_call(xp, w0s, deg)
    for k in range(_L - 1):
        sk = sc["spmm"](g, srcl, dstl, zwin)
        bk = b0 if k == 0 else bks[:, k - 1]
        g = _step_call(sk, deg, bk, wks[:, k])
    s9 = sc["spmm"](g, srcl, dstl, zwin)
    out = _final_call(s9, deg, bks[:, _L - 2], Watt)
    return out[:, :_N, :]


# speedup vs baseline: 3.7662x; 2.0386x over previous
import jax
import jax.numpy as jnp
from jax import lax
from jax.experimental import pallas as pl
from jax.experimental.pallas import tpu as pltpu
from jax.experimental.pallas import tpu_sc as plsc

_N = 10000
_E = 320000
_ETOT = _E + _N
_DIN = 128
_D = 64
_L = 10
_NC = 2
_NS = 16
_CHUNK = 128
_CHUNKS = 162
_EPAD = _NS * _CHUNKS * _CHUNK
_NW = 4
_WROWS = 2560
_NPAD = _NW * _WROWS
_ZR = _NPAD // _NS
_WRPT = _WROWS // _NS
_LCAPW = 5632
_LCAPWC = _LCAPW // _CHUNK
_LTOT = _NW * _LCAPW
_NBUF = 4


def _sc_bucket_body(src_hbm, dst_hbm, fsrc_hbm, fdst_hbm,
                    srcl_hbm, dstl_hbm, cnts_hbm,
                    src_v, dst_v, srcl_v, dstl_v, cbuf_v):
    c = lax.axis_index("c")
    s = lax.axis_index("s")
    w_id = c * _NS + s
    epw = _CHUNKS * _CHUNK
    pltpu.sync_copy(src_hbm.at[pl.ds(w_id * epw, epw)], src_v)
    pltpu.sync_copy(dst_hbm.at[pl.ds(w_id * epw, epw)], dst_v)
    pltpu.sync_copy(fsrc_hbm, srcl_v)
    pltpu.sync_copy(fdst_hbm, dstl_v)

    def scan_body(i, cnts):
        dstv = dst_v[pl.ds(i * 16, 16)]
        srcv = src_v[pl.ds(i * 16, 16)]
        new = []
        for w in range(_NW):
            lo = w * _WROWS
            m = (dstv >= lo) & (dstv < lo + _WROWS)
            csum = plsc.cumsum(m.astype(jnp.int32))
            pos = w * _LCAPW + cnts[w] + csum - 1
            plsc.store_scatter(srcl_v, [pos], srcv, mask=m)
            plsc.store_scatter(dstl_v, [pos], dstv - lo, mask=m)
            pc = plsc.all_reduce_population_count(m)
            new.append(jnp.minimum(cnts[w] + pc, _LCAPW - 16))
        return tuple(new)

    zero16 = jnp.zeros((16,), jnp.int32)
    cnts = lax.fori_loop(0, _CHUNKS * 8, scan_body,
                         (zero16, zero16, zero16, zero16))
    for w in range(_NW):
        cbuf_v[pl.ds(w * 16, 16)] = cnts[w]
    pltpu.sync_copy(srcl_v, srcl_hbm.at[pl.ds(w_id * _LTOT, _LTOT)])
    pltpu.sync_copy(dstl_v, dstl_hbm.at[pl.ds(w_id * _LTOT, _LTOT)])
    pltpu.sync_copy(cbuf_v, cnts_hbm.at[pl.ds(w_id * 64, 64)])


def _sc_spmm_body(g_hbm, srcl_hbm, dstl_hbm, z_hbm, out_hbm,
                  srcl_v, dstl_v, rows_v, zbuf_v, obuf_v, acc_sh, gsem, ssem):
    c = lax.axis_index("c")
    s = lax.axis_index("s")
    r0 = s * _WRPT
    pltpu.sync_copy(srcl_hbm.at[c, s], srcl_v)
    pltpu.sync_copy(dstl_hbm.at[c, s], dstl_v)
    pltpu.sync_copy(z_hbm, zbuf_v)
    g2 = g_hbm.at[c]
    gdum = g2.at[pl.ds(0, _CHUNK)]
    adum = acc_sh.at[pl.ds(0, _CHUNK)]
    pltpu.sync_copy(zbuf_v, acc_sh.at[pl.ds(r0, _WRPT)])
    plsc.subcore_barrier()
    nr = _LCAPWC // _NBUF
    for w in range(_NW):
        for b in range(_NBUF):
            pltpu.async_copy(g2.at[srcl_v.at[w, b]], rows_v.at[b], gsem.at[b])

        def round_body(r, carry):
            for b in range(_NBUF):
                pltpu.make_async_copy(gdum, rows_v.at[b], gsem.at[b]).wait()
                pltpu.async_copy(rows_v.at[b],
                                 acc_sh.at[dstl_v.at[w, r * _NBUF + b]],
                                 ssem.at[b], add=True)

            @pl.when(r + 1 < nr)
            def _():
                for b in range(_NBUF):
                    pltpu.make_async_copy(rows_v.at[b], adum, ssem.at[b]).wait()
                    pltpu.async_copy(g2.at[srcl_v.at[w, (r + 1) * _NBUF + b]],
                                     rows_v.at[b], gsem.at[b])

            return carry

        lax.fori_loop(0, nr, round_body, 0)
        for b in range(_NBUF):
            pltpu.make_async_copy(rows_v.at[b], adum, ssem.at[b]).wait()
        plsc.subcore_barrier()
        pltpu.sync_copy(acc_sh.at[pl.ds(r0, _WRPT)], obuf_v)
        pltpu.sync_copy(obuf_v, out_hbm.at[c].at[pl.ds(w * _WROWS + r0, _WRPT)])
        if w + 1 < _NW:
            pltpu.sync_copy(zbuf_v, acc_sh.at[pl.ds(r0, _WRPT)])
            plsc.subcore_barrier()


def _sc_deg_body(dstl_hbm, ones_hbm, z_hbm, out_hbm,
                 dstl_v, ones_v, zbuf_v, obuf_v, acc_sh, ssem):
    c = lax.axis_index("c")
    s = lax.axis_index("s")
    r0 = s * _WRPT
    pltpu.sync_copy(dstl_hbm.at[c, s], dstl_v)
    pltpu.sync_copy(ones_hbm, ones_v)
    pltpu.sync_copy(z_hbm, zbuf_v)
    adum = acc_sh.at[pl.ds(0, _CHUNK)]
    pltpu.sync_copy(zbuf_v, acc_sh.at[pl.ds(r0, _WRPT)])
    plsc.subcore_barrier()
    for w in range(_NW):

        def fire_body(j, carry):
            pltpu.async_copy(ones_v, acc_sh.at[dstl_v.at[w, j]], ssem, add=True)
            return carry

        lax.fori_loop(0, _LCAPWC, fire_body, 0)

        def drain_body(j, carry):
            pltpu.make_async_copy(ones_v, adum, ssem).wait()
            return carry

        lax.fori_loop(0, _LCAPWC, drain_body, 0)
        plsc.subcore_barrier()
        pltpu.sync_copy(acc_sh.at[pl.ds(r0, _WRPT)], obuf_v)
        pltpu.sync_copy(obuf_v, out_hbm.at[c].at[pl.ds(w * _WROWS + r0, _WRPT)])
        if w + 1 < _NW:
            pltpu.sync_copy(zbuf_v, acc_sh.at[pl.ds(r0, _WRPT)])
            plsc.subcore_barrier()


_sc_calls_cache = {}


def _sc_calls():
    if "bucket" not in _sc_calls_cache:
        mesh = plsc.VectorSubcoreMesh(core_axis_name="c", subcore_axis_name="s",
                                      num_cores=_NC, num_subcores=_NS)
        _sc_calls_cache["bucket"] = pl.kernel(
            _sc_bucket_body,
            out_type=(
                jax.ShapeDtypeStruct((_NC * _NS * _LTOT,), jnp.int32),
                jax.ShapeDtypeStruct((_NC * _NS * _LTOT,), jnp.int32),
                jax.ShapeDtypeStruct((_NC * _NS * 64,), jnp.int32),
            ),
            mesh=mesh,
            compiler_params=pltpu.CompilerParams(use_tc_tiling_on_sc=False, needs_layout_passes=False),
            scratch_types=[
                pltpu.VMEM((_CHUNKS * _CHUNK,), jnp.int32),
                pltpu.VMEM((_CHUNKS * _CHUNK,), jnp.int32),
                pltpu.VMEM((_LTOT,), jnp.int32),
                pltpu.VMEM((_LTOT,), jnp.int32),
                pltpu.VMEM((64,), jnp.int32),
            ],
        )
        _sc_calls_cache["deg"] = pl.kernel(
            _sc_deg_body,
            out_type=jax.ShapeDtypeStruct((_NC, _NPAD, _D), jnp.float32),
            mesh=mesh,
            compiler_params=pltpu.CompilerParams(use_tc_tiling_on_sc=False, needs_layout_passes=False),
            scratch_types=[
                pltpu.VMEM((_NW, _LCAPWC, _CHUNK), jnp.int32),
                pltpu.VMEM((_CHUNK, _D), jnp.float32),
                pltpu.VMEM((_WRPT, _D), jnp.float32),
                pltpu.VMEM((_WRPT, _D), jnp.float32),
                pltpu.VMEM_SHARED((_WROWS + 8, _D), jnp.float32),
                pltpu.SemaphoreType.DMA,
            ],
        )
        _sc_calls_cache["spmm"] = pl.kernel(
            _sc_spmm_body,
            out_type=jax.ShapeDtypeStruct((_NC, _NPAD, _D), jnp.float32),
            mesh=mesh,
            compiler_params=pltpu.CompilerParams(use_tc_tiling_on_sc=False, needs_layout_passes=False),
            scratch_types=[
                pltpu.VMEM((_NW, _LCAPWC, _CHUNK), jnp.int32),
                pltpu.VMEM((_NW, _LCAPWC, _CHUNK), jnp.int32),
                pltpu.VMEM((_NBUF, _CHUNK, _D), jnp.float32),
                pltpu.VMEM((_WRPT, _D), jnp.float32),
                pltpu.VMEM((_WRPT, _D), jnp.float32),
                pltpu.VMEM_SHARED((_WROWS + 8, _D), jnp.float32),
                pltpu.SemaphoreType.DMA((_NBUF,)),
                pltpu.SemaphoreType.DMA((_NBUF,)),
            ],
        )
    return _sc_calls_cache


def _rowmask():
    return lax.broadcasted_iota(jnp.int32, (_NPAD, 1), 0) < _N


def _dis(deg_ref, c):
    return lax.rsqrt(jnp.maximum(deg_ref[c, :, 0:1], 1.0))


def _tc_prep_body(xp_ref, w0_ref, deg_ref, g_ref):
    mask = _rowmask()
    for c in range(_NC):
        xw = jnp.dot(xp_ref[...], w0_ref[c], preferred_element_type=jnp.float32)
        g_ref[c] = jnp.where(mask, _dis(deg_ref, c) * xw, 0.0)


def _tc_step_body(s_ref, deg_ref, b_ref, w_ref, g_ref):
    mask = _rowmask()
    for c in range(_NC):
        dis = _dis(deg_ref, c)
        h = jnp.maximum(dis * s_ref[c] + b_ref[c], 0.0)
        g_ref[c] = jnp.where(
            mask, dis * jnp.dot(h, w_ref[c], preferred_element_type=jnp.float32), 0.0)


def _tc_final_body(s_ref, deg_ref, b_ref, watt_ref, out_ref):
    feats = []
    for c in range(_NC):
        feats.append(_dis(deg_ref, c) * s_ref[c] + b_ref[c])
    nf = jnp.concatenate(feats, axis=1)
    nrm = lax.rsqrt(jnp.sum(nf * nf, axis=1, keepdims=True))
    nfn = nf * nrm
    mask = _rowmask()
    nfn_m = jnp.where(mask, nfn, 0.0)
    mean = jnp.sum(nfn_m, axis=0, keepdims=True) * (1.0 / _N)
    ctx = jnp.tanh(jnp.dot(mean, watt_ref[...], preferred_element_type=jnp.float32))
    score = jax.nn.sigmoid(jnp.sum(nfn_m * ctx, axis=1, keepdims=True))
    gf = jnp.sum(jnp.where(mask, score * nfn_m, 0.0), axis=0, keepdims=True)
    out_ref[0] = jnp.concatenate(
        [nfn_m, jnp.broadcast_to(gf, (_NPAD, 2 * _D))], axis=1)


def _prep_call(xp, w0s, deg):
    return pl.pallas_call(
        _tc_prep_body,
        out_shape=jax.ShapeDtypeStruct((_NC, _NPAD, _D), jnp.float32),
    )(xp, w0s, deg)


def _step_call(sk, deg, bk, wk):
    return pl.pallas_call(
        _tc_step_body,
        out_shape=jax.ShapeDtypeStruct((_NC, _NPAD, _D), jnp.float32),
    )(sk, deg, bk, wk)


def _final_call(s9, deg, b9, watt):
    return pl.pallas_call(
        _tc_final_body,
        out_shape=jax.ShapeDtypeStruct((1, _NPAD, 4 * _D), jnp.float32),
    )(s9, deg, b9, watt)


def kernel(x, edge_index, batch, Wf0, bf0, Wf, bf, Wr0, br0, Wr, br, Watt):
    loopv = jnp.arange(_N, dtype=jnp.int32)
    padv = jnp.full((_EPAD - _ETOT,), jnp.int32(1 << 30), jnp.int32)
    a = jnp.concatenate([edge_index[0], loopv, padv])
    b = jnp.concatenate([edge_index[1], loopv, padv])
    src2 = jnp.stack([a, b]).reshape(-1)
    dst2 = jnp.stack([b, a]).reshape(-1)

    fsrc = jnp.full((_LTOT,), _N, jnp.int32)
    fdst = jnp.full((_LTOT,), _WROWS, jnp.int32)
    zwin = jnp.zeros((_WRPT, _D), jnp.float32)
    ones128 = jnp.ones((_CHUNK, _D), jnp.float32)
    xp = jnp.pad(x, ((0, _NPAD - _N), (0, 0)))

    w0s = jnp.stack([Wf0, Wr0])
    wks = jnp.stack([Wf, Wr])
    b0 = jnp.stack([bf0, br0])
    bks = jnp.stack([bf, br])

    sc = _sc_calls()
    srcl, dstl, cnts = sc["bucket"](src2, dst2, fsrc, fdst)
    srcl = srcl.reshape(_NC, _NS, _NW, _LCAPWC, _CHUNK)
    dstl = dstl.reshape(_NC, _NS, _NW, _LCAPWC, _CHUNK)

    deg = sc["deg"](dstl, ones128, zwin)
    g = _prep_call(xp, w0s, deg)
    for k in range(_L - 1):
        sk = sc["spmm"](g, srcl, dstl, zwin)
        bk = b0 if k == 0 else bks[:, k - 1]
        g = _step_call(sk, deg, bk, wks[:, k])
    s9 = sc["spmm"](g, srcl, dstl, zwin)
    out = _final_call(s9, deg, bks[:, _L - 2], Watt)
    return out[:, :_N, :]
